# baseline (speedup 1.0000x reference)
import jax
import jax.numpy as jnp
from jax import lax
from jax.experimental import pallas as pl
from jax.experimental.pallas import tpu as pltpu
from jax.experimental.pallas import tpu_sc as plsc

B = 4096
L = 50
E = 128
H = 512
O = 128

NC = 2
NS = 16
NW = NC * NS
EV = E // 16
UNROLL = 1
NBUF = 12


def _make_bag(nrows):
    cpw = nrows // NW

    def body(idx_hbm, table_hbm, out_hbm, idx_v, gbufs, acc, sems):
        wid = lax.axis_index("s") * NC + lax.axis_index("c")
        pltpu.sync_copy(idx_hbm.at[pl.ds(wid * cpw, cpw)], idx_v)

        def start(c, b):
            pltpu.make_async_copy(table_hbm.at[idx_v.at[c]], gbufs.at[b],
                                  sems.at[b]).start()

        def wait(b):
            pltpu.make_async_copy(table_hbm.at[idx_v.at[0]], gbufs.at[b],
                                  sems.at[b]).wait()

        def accum(b, c):
            def add_body(j, vecs):
                return tuple(
                    vecs[k] + gbufs[b, j, pl.ds(16 * k, 16)]
                    for k in range(EV))

            init = tuple(jnp.zeros((16,), jnp.float32) for _ in range(EV))
            vecs = lax.fori_loop(0, L, add_body, init)
            for k in range(EV):
                acc[c, pl.ds(16 * k, 16)] = vecs[k]

        for b in range(NBUF):
            start(b, b)

        def loop_body(c, carry):
            b = lax.rem(c, NBUF)
            wait(b)
            accum(b, c)
            start(lax.rem(c + NBUF, cpw), b)
            return carry

        lax.fori_loop(0, cpw, loop_body, 0)
        for b in range(NBUF):
            wait(b)
        pltpu.sync_copy(acc, out_hbm.at[pl.ds(wid * cpw, cpw)])

    return pl.kernel(
        body,
        mesh=plsc.VectorSubcoreMesh(core_axis_name="c",
                                    subcore_axis_name="s"),
        out_type=jax.ShapeDtypeStruct((nrows, E), jnp.float32),
        scratch_types=(
            [pltpu.VMEM((cpw, L), jnp.int32)]
            + [pltpu.VMEM((NBUF, L, E), jnp.float32)]
            + [pltpu.VMEM((cpw, E), jnp.float32)]
            + [pltpu.SemaphoreType.DMA((NBUF,))]
        ),
    )


BM = 4096


def _mlp_body(x_ref, w1_ref, b1_ref, w2_ref, b2_ref, w3_ref, b3_ref, o_ref):
    x = x_ref[...].astype(jnp.bfloat16)
    h = jnp.tanh(jnp.dot(x, w1_ref[...],
                         preferred_element_type=jnp.float32) + b1_ref[...])
    h = jnp.tanh(jnp.dot(h.astype(jnp.bfloat16), w2_ref[...],
                         preferred_element_type=jnp.float32) + b2_ref[...])
    o_ref[...] = jnp.dot(h.astype(jnp.bfloat16), w3_ref[...],
                         preferred_element_type=jnp.float32) + b3_ref[...]


def _make_mlp(nrows):
    return pl.pallas_call(
        _mlp_body,
        grid=(nrows // BM,),
        in_specs=[
            pl.BlockSpec((BM, E), lambda i: (i, 0)),
            pl.BlockSpec((E, H), lambda i: (0, 0)),
            pl.BlockSpec((1, H), lambda i: (0, 0)),
            pl.BlockSpec((H, H), lambda i: (0, 0)),
            pl.BlockSpec((1, H), lambda i: (0, 0)),
            pl.BlockSpec((H, O), lambda i: (0, 0)),
            pl.BlockSpec((1, O), lambda i: (0, 0)),
        ],
        out_specs=pl.BlockSpec((BM, O), lambda i: (i, 0)),
        out_shape=jax.ShapeDtypeStruct((nrows, O), jnp.float32),
    )


def kernel(inputs, embed, W1, b1, W2, b2, W3, b3):
    x = _make_bag(B)(inputs, embed)
    return _make_mlp(B)(x, W1.astype(jnp.bfloat16), b1.reshape(1, H),
                        W2.astype(jnp.bfloat16), b2.reshape(1, H),
                        W3.astype(jnp.bfloat16), b3.reshape(1, O))


# speedup vs baseline: 1.0865x; 1.0865x over previous
import jax
import jax.numpy as jnp
from jax import lax
from jax.experimental import pallas as pl
from jax.experimental.pallas import tpu as pltpu
from jax.experimental.pallas import tpu_sc as plsc

B = 4096
L = 50
E = 128
H = 512
O = 128

NC = 2
NS = 16
NW = NC * NS
EV = E // 16
UNROLL = 1
NBUF = 8


def _make_bag(nrows):
    cpw = nrows // NW

    def body(idx_hbm, table_hbm, out_hbm, idx_v, gbufs, acc, sems):
        wid = lax.axis_index("s") * NC + lax.axis_index("c")
        pltpu.sync_copy(idx_hbm.at[pl.ds(wid * cpw, cpw)], idx_v)

        def start(c, b):
            pltpu.make_async_copy(table_hbm.at[idx_v.at[c]], gbufs.at[b],
                                  sems.at[b]).start()

        def wait(b):
            pltpu.make_async_copy(table_hbm.at[idx_v.at[0]], gbufs.at[b],
                                  sems.at[b]).wait()

        def accum(b, c):
            def add_body(j, vecs):
                return tuple(
                    vecs[k] + gbufs[b, j, pl.ds(16 * k, 16)]
                    for k in range(EV))

            init = tuple(jnp.zeros((16,), jnp.float32) for _ in range(EV))
            vecs = lax.fori_loop(0, L, add_body, init)
            for k in range(EV):
                acc[c, pl.ds(16 * k, 16)] = vecs[k]

        for b in range(NBUF):
            start(b, b)

        def loop_body(c, carry):
            b = lax.rem(c, NBUF)
            wait(b)
            accum(b, c)
            start(lax.rem(c + NBUF, cpw), b)
            return carry

        lax.fori_loop(0, cpw, loop_body, 0)
        for b in range(NBUF):
            wait(b)
        pltpu.sync_copy(acc, out_hbm.at[pl.ds(wid * cpw, cpw)])

    return pl.kernel(
        body,
        mesh=plsc.VectorSubcoreMesh(core_axis_name="c",
                                    subcore_axis_name="s"),
        out_type=jax.ShapeDtypeStruct((nrows, E), jnp.float32),
        scratch_types=(
            [pltpu.VMEM((cpw, L), jnp.int32)]
            + [pltpu.VMEM((NBUF, L, E), jnp.float32)]
            + [pltpu.VMEM((cpw, E), jnp.float32)]
            + [pltpu.SemaphoreType.DMA((NBUF,))]
        ),
    )


BM = 4096


def _mlp_body(x_ref, w1_ref, b1_ref, w2_ref, b2_ref, w3_ref, b3_ref, o_ref):
    x = x_ref[...].astype(jnp.bfloat16)
    h = jnp.tanh(jnp.dot(x, w1_ref[...],
                         preferred_element_type=jnp.float32) + b1_ref[...])
    h = jnp.tanh(jnp.dot(h.astype(jnp.bfloat16), w2_ref[...],
                         preferred_element_type=jnp.float32) + b2_ref[...])
    o_ref[...] = jnp.dot(h.astype(jnp.bfloat16), w3_ref[...],
                         preferred_element_type=jnp.float32) + b3_ref[...]


def _make_mlp(nrows):
    return pl.pallas_call(
        _mlp_body,
        grid=(nrows // BM,),
        in_specs=[
            pl.BlockSpec((BM, E), lambda i: (i, 0)),
            pl.BlockSpec((E, H), lambda i: (0, 0)),
            pl.BlockSpec((1, H), lambda i: (0, 0)),
            pl.BlockSpec((H, H), lambda i: (0, 0)),
            pl.BlockSpec((1, H), lambda i: (0, 0)),
            pl.BlockSpec((H, O), lambda i: (0, 0)),
            pl.BlockSpec((1, O), lambda i: (0, 0)),
        ],
        out_specs=pl.BlockSpec((BM, O), lambda i: (i, 0)),
        out_shape=jax.ShapeDtypeStruct((nrows, O), jnp.float32),
    )


def kernel(inputs, embed, W1, b1, W2, b2, W3, b3):
    x = _make_bag(B)(inputs, embed)
    return _make_mlp(B)(x, W1.astype(jnp.bfloat16), b1.reshape(1, H),
                        W2.astype(jnp.bfloat16), b2.reshape(1, H),
                        W3.astype(jnp.bfloat16), b3.reshape(1, O))


# speedup vs baseline: 1.0881x; 1.0015x over previous
import jax
import jax.numpy as jnp
from jax import lax
from jax.experimental import pallas as pl
from jax.experimental.pallas import tpu as pltpu
from jax.experimental.pallas import tpu_sc as plsc

B = 4096
L = 50
E = 128
H = 512
O = 128

NC = 2
NS = 16
NW = NC * NS
EV = E // 16
NBUF = 8


def _make_bag(nrows):
    cpw = nrows // NW

    def body(idx_hbm, table_hbm, out_hbm, idx_v, gbufs, acc, sems):
        wid = lax.axis_index("s") * NC + lax.axis_index("c")
        pltpu.sync_copy(idx_hbm.at[pl.ds(wid * cpw, cpw)], idx_v)

        def start(c, b):
            pltpu.make_async_copy(table_hbm.at[idx_v.at[c]], gbufs.at[b],
                                  sems.at[b]).start()

        def wait(b):
            pltpu.make_async_copy(table_hbm.at[idx_v.at[0]], gbufs.at[b],
                                  sems.at[b]).wait()

        def accum(b, c):
            def add_body(j, vecs):
                return tuple(
                    vecs[k] + gbufs[b, j, pl.ds(16 * k, 16)]
                    for k in range(EV))

            init = tuple(jnp.zeros((16,), jnp.float32) for _ in range(EV))
            vecs = lax.fori_loop(0, L, add_body, init)
            for k in range(EV):
                acc[c, pl.ds(16 * k, 16)] = vecs[k]

        for b in range(NBUF):
            start(b, b)

        def loop_body(c, carry):
            b = lax.rem(c, NBUF)
            wait(b)
            accum(b, c)
            start(lax.rem(c + NBUF, cpw), b)
            return carry

        lax.fori_loop(0, cpw, loop_body, 0)
        for b in range(NBUF):
            wait(b)
        pltpu.sync_copy(acc, out_hbm.at[pl.ds(wid * cpw, cpw)])

    return pl.kernel(
        body,
        mesh=plsc.VectorSubcoreMesh(core_axis_name="c",
                                    subcore_axis_name="s"),
        out_type=jax.ShapeDtypeStruct((nrows, E), jnp.float32),
        scratch_types=(
            [pltpu.VMEM((cpw, L), jnp.int32)]
            + [pltpu.VMEM((NBUF, L, E), jnp.float32)]
            + [pltpu.VMEM((cpw, E), jnp.float32)]
            + [pltpu.SemaphoreType.DMA((NBUF,))]
        ),
    )


BM = 4096


def _mlp_body(x_ref, w1_ref, b1_ref, w2_ref, b2_ref, w3_ref, b3_ref, o_ref):
    x = x_ref[...].astype(jnp.bfloat16)
    h = jnp.tanh(jnp.dot(x, w1_ref[...],
                         preferred_element_type=jnp.float32) + b1_ref[...])
    h = jnp.tanh(jnp.dot(h.astype(jnp.bfloat16), w2_ref[...],
                         preferred_element_type=jnp.float32) + b2_ref[...])
    o_ref[...] = jnp.dot(h.astype(jnp.bfloat16), w3_ref[...],
                         preferred_element_type=jnp.float32) + b3_ref[...]


def _make_mlp(nrows):
    return pl.pallas_call(
        _mlp_body,
        grid=(nrows // BM,),
        in_specs=[
            pl.BlockSpec((BM, E), lambda i: (i, 0)),
            pl.BlockSpec((E, H), lambda i: (0, 0)),
            pl.BlockSpec((1, H), lambda i: (0, 0)),
            pl.BlockSpec((H, H), lambda i: (0, 0)),
            pl.BlockSpec((1, H), lambda i: (0, 0)),
            pl.BlockSpec((H, O), lambda i: (0, 0)),
            pl.BlockSpec((1, O), lambda i: (0, 0)),
        ],
        out_specs=pl.BlockSpec((BM, O), lambda i: (i, 0)),
        out_shape=jax.ShapeDtypeStruct((nrows, O), jnp.float32),
    )


def kernel(inputs, embed, W1, b1, W2, b2, W3, b3):
    x = _make_bag(B)(inputs, embed)
    return _make_mlp(B)(x, W1.astype(jnp.bfloat16), b1.reshape(1, H),
                        W2.astype(jnp.bfloat16), b2.reshape(1, H),
                        W3.astype(jnp.bfloat16), b3.reshape(1, O))
